# baseline (speedup 1.0000x reference)
import functools

import jax
import jax.numpy as jnp
from jax import lax
from jax.experimental import pallas as pl
from jax.experimental.pallas import tpu as pltpu
from jax.experimental.pallas import tpu_sc as plsc

NC = 2
NS = 16
L = 16
NW = NC * NS


def _make_scale_kernel(B, V, H, W, C, inv):
  ROWS = B * V
  assert ROWS % NW == 0
  rpw = ROWS // NW
  assert rpw % L == 0
  n_chunks = rpw // L

  mesh = plsc.VectorSubcoreMesh(
      core_axis_name="c", subcore_axis_name="s",
      num_cores=NC, num_subcores=NS)

  scratch = [
      pltpu.VMEM((rpw,), jnp.float32),
      pltpu.VMEM((rpw,), jnp.float32),
      [pltpu.VMEM((4 * L,), jnp.int32) for _ in range(2)],
      [pltpu.VMEM((4 * L, C), jnp.float32) for _ in range(2)],
      [pltpu.VMEM((L, C), jnp.float32) for _ in range(2)],
      [pltpu.SemaphoreType.DMA for _ in range(2)],
      [pltpu.SemaphoreType.DMA for _ in range(2)],
  ]

  @functools.partial(
      pl.kernel,
      mesh=mesh,
      out_type=jax.ShapeDtypeStruct((ROWS, C), jnp.float32),
      scratch_types=scratch,
  )
  def k(cx_hbm, cy_hbm, t_hbm, out_hbm,
        cx_v, cy_v, idx_vs, gbufs, obufs, gsems, osems):
    wid = lax.axis_index("s") * NC + lax.axis_index("c")
    base = wid * rpw
    batch = base // V

    pltpu.sync_copy(cx_hbm.at[pl.ds(base, rpw)], cx_v)
    pltpu.sync_copy(cy_hbm.at[pl.ds(base, rpw)], cy_v)

    def corner_geom(ch):
      x = cx_v[pl.ds(ch * L, L)] * inv
      y = cy_v[pl.ds(ch * L, L)] * inv
      x1i = x.astype(jnp.int32)
      y1i = y.astype(jnp.int32)
      x1f = x1i.astype(jnp.float32)
      y1f = y1i.astype(jnp.float32)
      one = jnp.full((L,), 1, jnp.int32)
      zero = jnp.full((L,), 0, jnp.int32)
      x2i = x1i + jnp.where(x > x1f, one, zero)
      y2i = y1i + jnp.where(y > y1f, one, zero)
      return x, y, x1i, y1i, x1f, y1f, x2i, y2i

    def fire(ch, slot):
      _, _, x1i, y1i, _, _, x2i, y2i = corner_geom(ch)
      idx_v = idx_vs[slot]
      r1 = y1i * W + batch * (H * W)
      r2 = y2i * W + batch * (H * W)
      idx_v[pl.ds(0 * L, L)] = r1 + x1i
      idx_v[pl.ds(1 * L, L)] = r2 + x1i
      idx_v[pl.ds(2 * L, L)] = r1 + x2i
      idx_v[pl.ds(3 * L, L)] = r2 + x2i
      return pltpu.async_copy(t_hbm.at[idx_v], gbufs[slot], gsems[slot])

    dn = lax.GatherDimensionNumbers(
        offset_dims=(), collapsed_slice_dims=(0,), start_index_map=(0,))

    def splat(vec, sp):
      return lax.gather(vec, sp[:, None], dn, (1,),
                        mode=lax.GatherScatterMode.PROMISE_IN_BOUNDS)

    def compute(ch, slot):
      x, y, _, _, x1f, y1f, x2i, y2i = corner_geom(ch)
      wx2 = x - x1f
      wx1 = x2i.astype(jnp.float32) - x
      wy2 = y - y1f
      wy1 = y2i.astype(jnp.float32) - y
      w11 = wx1 * wy1
      w12 = wx1 * wy2
      w21 = wx2 * wy1
      w22 = wx2 * wy2
      gbuf = gbufs[slot]
      obuf = obufs[slot]

      @plsc.parallel_loop(0, L)
      def row_body(r):
        sp = jnp.full((L,), 0, jnp.int32) + r
        w0 = splat(w11, sp)
        w1 = splat(w12, sp)
        w2 = splat(w21, sp)
        w3 = splat(w22, sp)

        @plsc.parallel_loop(0, C // L, unroll=4)
        def ch_body(j):
          acc = w0 * gbuf[0 * L + r, pl.ds(j * L, L)]
          acc += w1 * gbuf[1 * L + r, pl.ds(j * L, L)]
          acc += w2 * gbuf[2 * L + r, pl.ds(j * L, L)]
          acc += w3 * gbuf[3 * L + r, pl.ds(j * L, L)]
          obuf[r, pl.ds(j * L, L)] = acc

    handles = [fire(0, 0)]
    owrites = [None] * n_chunks
    for ch in range(n_chunks):
      slot = ch % 2
      if ch + 1 < n_chunks:
        handles.append(fire(ch + 1, (ch + 1) % 2))
      handles[ch].wait()
      if ch >= 2 and owrites[ch - 2] is not None:
        owrites[ch - 2].wait()
      compute(ch, slot)
      owrites[ch] = pltpu.async_copy(
          obufs[slot], out_hbm.at[pl.ds(base + ch * L, L)], osems[slot])
    for ch in range(max(0, n_chunks - 2), n_chunks):
      if owrites[ch] is not None:
        owrites[ch].wait()

  return k


def kernel(c, conv_3_3, conv_4_3, conv_5_3):
  B, V, _ = c.shape
  maps = (conv_3_3, conv_4_3, conv_5_3)

  cx = c[:, :, 0].reshape(-1)
  cy = c[:, :, 1].reshape(-1)

  outs = []
  inv = 1.0 / 8.0
  for fm in maps:
    _, C, H, W = fm.shape
    table = fm.transpose(0, 2, 3, 1).reshape(-1, C)
    k = _make_scale_kernel(B, V, H, W, C, inv)
    outs.append(k(cx, cy, table).reshape(B, V, C))
    inv *= 0.5

  return jnp.concatenate(outs, axis=2)
